# Initial kernel scaffold; baseline (speedup 1.0000x reference)
import jax
import jax.numpy as jnp
from jax.experimental import pallas as pl


def kernel(x, edge_index, W, b):
    raise NotImplementedError("write your pallas kernel here")


# speedup vs baseline: 13.1079x; 13.1079x over previous
import functools

import jax
import jax.numpy as jnp
from jax import lax
from jax.experimental import pallas as pl
from jax.experimental.pallas import tpu as pltpu
from jax.experimental.pallas import tpu_sc as plsc

N = 10000
E = 320000
D_IN = 128
D_MODEL = 256
H = D_MODEL // 2

NC = 2
NS = 16
NW = NC * NS

NP = 10240
ECW = E // NW
ECS = E // NS
B = 80
NB = ECS // B
R = NP // NS
RCH = 128
BN = 1000

_mesh = plsc.VectorSubcoreMesh(core_axis_name="c", subcore_axis_name="s")


def _hist_body(dst_hbm, out_hbm, dst_v, hist_v):
    c = lax.axis_index("c")
    s = lax.axis_index("s")
    wid = s * NC + c

    def zero(i, _):
        hist_v[pl.ds(pl.multiple_of(i * 16, 16), 16)] = jnp.zeros((16,), jnp.float32)
        return 0

    lax.fori_loop(0, N // 16, zero, 0)

    pltpu.sync_copy(dst_hbm.at[pl.ds(wid * ECW, ECW)], dst_v)

    ones = jnp.full((16,), 1.0, jnp.float32)

    def add(i, _):
        idx = dst_v[pl.ds(pl.multiple_of(i * 16, 16), 16)]
        plsc.addupdate_scatter(hist_v, [idx], ones)
        return 0

    lax.fori_loop(0, ECW // 16, add, 0)
    pltpu.sync_copy(hist_v, out_hbm.at[wid])


_k1 = functools.partial(
    pl.kernel,
    out_type=jax.ShapeDtypeStruct((NW, N), jnp.float32),
    mesh=_mesh,
    scratch_types=[
        pltpu.VMEM((ECW,), jnp.int32),
        pltpu.VMEM((N,), jnp.float32),
    ],
    compiler_params=pltpu.CompilerParams(needs_layout_passes=False),
)(_hist_body)


def _mm_body(x_ref, part_ref, w_ref, g_ref, dinv_ref):
    deg = jnp.sum(part_ref[...], axis=1) + 1.0
    dinv = lax.rsqrt(deg)
    h = jnp.dot(x_ref[...], w_ref[...], preferred_element_type=jnp.float32)
    g = h * dinv[:, None]
    g_ref[...] = jnp.stack([g[:, :H], g[:, H:]], axis=0)
    dinv_ref[...] = dinv[:, None]


def _k2(x, part, W):
    return pl.pallas_call(
        _mm_body,
        grid=(N // BN,),
        in_specs=[
            pl.BlockSpec((BN, D_IN), lambda i: (i, 0)),
            pl.BlockSpec((BN, NW), lambda i: (i, 0)),
            pl.BlockSpec((D_IN, D_MODEL), lambda i: (0, 0)),
        ],
        out_specs=[
            pl.BlockSpec((2, BN, H), lambda i: (0, i, 0)),
            pl.BlockSpec((BN, 1), lambda i: (i, 0)),
        ],
        out_shape=[
            jax.ShapeDtypeStruct((2, NP, H), jnp.float32),
            jax.ShapeDtypeStruct((N, 1), jnp.float32),
        ],
    )(x, part, W)


def _scatter_body(src_hbm, dst_hbm, g_hbm, out_hbm, src_v, dst_v, rows_v, s_sh, sem):
    c = lax.axis_index("c")
    s = lax.axis_index("s")
    cn = c * NP

    def init(i, _):
        r0 = s * R + i * RCH
        pltpu.sync_copy(g_hbm.at[pl.ds(cn + r0, RCH)], s_sh.at[pl.ds(r0, RCH)])
        return 0

    lax.fori_loop(0, R // RCH, init, 0)
    plsc.subcore_barrier()

    base_e = s * ECS

    def blk(k, _):
        off = base_e + k * B
        pltpu.sync_copy(src_hbm.at[pl.ds(off, B)], src_v)
        pltpu.sync_copy(dst_hbm.at[pl.ds(off, B)], dst_v)

        def adj(j, _):
            sl = pl.ds(pl.multiple_of(j * 16, 16), 16)
            src_v[sl] = src_v[sl] + cn
            return 0

        lax.fori_loop(0, B // 16, adj, 0)
        pltpu.async_copy(g_hbm.at[src_v], rows_v, sem).wait()
        pltpu.sync_copy(rows_v, s_sh.at[dst_v], add=True)
        return 0

    lax.fori_loop(0, NB, blk, 0)
    plsc.subcore_barrier()

    def wb(i, _):
        r0 = s * R + i * RCH
        pltpu.sync_copy(s_sh.at[pl.ds(r0, RCH)], out_hbm.at[pl.ds(cn + r0, RCH)])
        return 0

    lax.fori_loop(0, R // RCH, wb, 0)


_k3 = functools.partial(
    pl.kernel,
    out_type=jax.ShapeDtypeStruct((2 * NP, H), jnp.float32),
    mesh=_mesh,
    scratch_types=[
        pltpu.VMEM((B,), jnp.int32),
        pltpu.VMEM((B,), jnp.int32),
        pltpu.VMEM((B, H), jnp.float32),
        pltpu.VMEM_SHARED((NP, H), jnp.float32),
        pltpu.SemaphoreType.DMA,
    ],
    compiler_params=pltpu.CompilerParams(needs_layout_passes=False),
)(_scatter_body)


def _ep_body(s0_ref, s1_ref, dinv_ref, b_ref, out_ref):
    m = jnp.concatenate([s0_ref[...], s1_ref[...]], axis=1)
    out_ref[...] = (m * dinv_ref[...] + b_ref[...])[None]


def _k4(s0, s1, dinv, b2):
    return pl.pallas_call(
        _ep_body,
        grid=(N // BN,),
        in_specs=[
            pl.BlockSpec((BN, H), lambda i: (i, 0)),
            pl.BlockSpec((BN, H), lambda i: (i, 0)),
            pl.BlockSpec((BN, 1), lambda i: (i, 0)),
            pl.BlockSpec((1, D_MODEL), lambda i: (0, 0)),
        ],
        out_specs=pl.BlockSpec((1, BN, D_MODEL), lambda i: (0, i, 0)),
        out_shape=jax.ShapeDtypeStruct((1, N, D_MODEL), jnp.float32),
    )(s0, s1, dinv, b2)


def kernel(x, edge_index, W, b):
    src = edge_index[0]
    dst = edge_index[1]
    part = _k1(dst)
    g2, dinv = _k2(x, part.T, W)
    g_cat = g2.reshape(2 * NP, H)
    s_cat = _k3(src, dst, g_cat)
    out = _k4(s_cat[:N], s_cat[NP:NP + N], dinv, b.reshape(1, D_MODEL))
    return (out, None, out)


# speedup vs baseline: 26.9373x; 2.0550x over previous
import functools

import jax
import jax.numpy as jnp
from jax import lax
from jax.experimental import pallas as pl
from jax.experimental.pallas import tpu as pltpu
from jax.experimental.pallas import tpu_sc as plsc

N = 10000
E = 320000
D_IN = 128
D_MODEL = 256
H = D_MODEL // 2

NC = 2
NS = 16
NW = NC * NS

NP = 10240
ECW = E // NW
B = 125
RT = (E // B) // NS
PAIRS = RT // 2
R = NP // NS
RCH = 128
BN = 1000

_mesh = plsc.VectorSubcoreMesh(core_axis_name="c", subcore_axis_name="s")


def _hist_body(dst_hbm, out_hbm, dst_v, hist_v):
    c = lax.axis_index("c")
    s = lax.axis_index("s")
    wid = s * NC + c

    def zero(i, _):
        hist_v[pl.ds(pl.multiple_of(i * 16, 16), 16)] = jnp.zeros((16,), jnp.float32)
        return 0

    lax.fori_loop(0, N // 16, zero, 0)

    pltpu.sync_copy(dst_hbm.at[pl.ds(wid * ECW, ECW)], dst_v)

    ones = jnp.full((16,), 1.0, jnp.float32)

    def add(i, _):
        idx = dst_v[pl.ds(pl.multiple_of(i * 16, 16), 16)]
        plsc.addupdate_scatter(hist_v, [idx], ones)
        return 0

    lax.fori_loop(0, ECW // 16, add, 0)
    pltpu.sync_copy(hist_v, out_hbm.at[wid])


_k1 = functools.partial(
    pl.kernel,
    out_type=jax.ShapeDtypeStruct((NW, N), jnp.float32),
    mesh=_mesh,
    scratch_types=[
        pltpu.VMEM((ECW,), jnp.int32),
        pltpu.VMEM((N,), jnp.float32),
    ],
    compiler_params=pltpu.CompilerParams(needs_layout_passes=False),
)(_hist_body)


def _mm_body(x_ref, part_ref, w_ref, g0_ref, g1_ref, dinv_ref):
    deg = jnp.sum(part_ref[...], axis=1) + 1.0
    dinv = lax.rsqrt(deg)
    h = jnp.dot(x_ref[...], w_ref[...], preferred_element_type=jnp.float32)
    g = h * dinv[:, None]
    g0_ref[...] = g[:, :H]
    g1_ref[...] = g[:, H:]
    dinv_ref[...] = dinv[:, None]


def _k2(x, part, W):
    return pl.pallas_call(
        _mm_body,
        grid=(N // BN,),
        in_specs=[
            pl.BlockSpec((BN, D_IN), lambda i: (i, 0)),
            pl.BlockSpec((BN, NW), lambda i: (i, 0)),
            pl.BlockSpec((D_IN, D_MODEL), lambda i: (0, 0)),
        ],
        out_specs=[
            pl.BlockSpec((BN, H), lambda i: (i, 0)),
            pl.BlockSpec((BN, H), lambda i: (i, 0)),
            pl.BlockSpec((BN, 1), lambda i: (i, 0)),
        ],
        out_shape=[
            jax.ShapeDtypeStruct((NP, H), jnp.float32),
            jax.ShapeDtypeStruct((NP, H), jnp.float32),
            jax.ShapeDtypeStruct((N, 1), jnp.float32),
        ],
    )(x, part, W)


def _scatter_body(ei2_hbm, g0_hbm, g1_hbm, out0_hbm, out1_hbm,
                  e0, e1, rows0, rows1, s_sh, es0, es1, gs0, gs1):
    c = lax.axis_index("c")
    s = lax.axis_index("s")
    base = s * RT

    def fetch_idx(k, buf, sem):
        pltpu.async_copy(ei2_hbm.at[base + k], buf, sem)

    def wait_idx(buf, sem):
        pltpu.make_async_copy(ei2_hbm.at[0], buf, sem).wait()

    fetch_idx(0, e0, es0)
    fetch_idx(1, e1, es1)

    def init(i, _):
        q = s * R + i * RCH
        sl = pl.ds(q, RCH)

        @pl.when(c == 0)
        def _():
            pltpu.sync_copy(g0_hbm.at[sl], s_sh.at[sl])

        @pl.when(c == 1)
        def _():
            pltpu.sync_copy(g1_hbm.at[sl], s_sh.at[sl])

        return 0

    lax.fori_loop(0, R // RCH, init, 0)
    plsc.subcore_barrier()

    def start_gather(ebuf, buf, sem):
        @pl.when(c == 0)
        def _():
            pltpu.async_copy(g0_hbm.at[ebuf.at[0]], buf, sem)

        @pl.when(c == 1)
        def _():
            pltpu.async_copy(g1_hbm.at[ebuf.at[0]], buf, sem)

    def wait_gather(buf, sem):
        pltpu.make_async_copy(g0_hbm.at[e0.at[0]], buf, sem).wait()

    def scat(ebuf, buf):
        pltpu.sync_copy(buf, s_sh.at[ebuf.at[1]], add=True)

    wait_idx(e0, es0)
    start_gather(e0, rows0, gs0)

    def body(p, _):
        k0 = 2 * p
        wait_idx(e1, es1)
        start_gather(e1, rows1, gs1)
        wait_gather(rows0, gs0)
        scat(e0, rows0)

        @pl.when(p + 1 < PAIRS)
        def _():
            fetch_idx(k0 + 2, e0, es0)

        wait_gather(rows1, gs1)
        scat(e1, rows1)

        @pl.when(p + 1 < PAIRS)
        def _():
            wait_idx(e0, es0)
            start_gather(e0, rows0, gs0)
            fetch_idx(k0 + 3, e1, es1)

        return 0

    lax.fori_loop(0, PAIRS, body, 0)
    plsc.subcore_barrier()

    def wb(i, _):
        q = s * R + i * RCH
        sl = pl.ds(q, RCH)

        @pl.when(c == 0)
        def _():
            pltpu.sync_copy(s_sh.at[sl], out0_hbm.at[sl])

        @pl.when(c == 1)
        def _():
            pltpu.sync_copy(s_sh.at[sl], out1_hbm.at[sl])

        return 0

    lax.fori_loop(0, R // RCH, wb, 0)


_k3 = functools.partial(
    pl.kernel,
    out_type=[
        jax.ShapeDtypeStruct((NP, H), jnp.float32),
        jax.ShapeDtypeStruct((NP, H), jnp.float32),
    ],
    mesh=_mesh,
    scratch_types=[
        pltpu.VMEM((2, B), jnp.int32),
        pltpu.VMEM((2, B), jnp.int32),
        pltpu.VMEM((B, H), jnp.float32),
        pltpu.VMEM((B, H), jnp.float32),
        pltpu.VMEM_SHARED((NP, H), jnp.float32),
        pltpu.SemaphoreType.DMA,
        pltpu.SemaphoreType.DMA,
        pltpu.SemaphoreType.DMA,
        pltpu.SemaphoreType.DMA,
    ],
    compiler_params=pltpu.CompilerParams(needs_layout_passes=False),
)(_scatter_body)


def _ep_body(s0_ref, s1_ref, dinv_ref, b_ref, out_ref):
    m = jnp.concatenate([s0_ref[...], s1_ref[...]], axis=1)
    out_ref[...] = (m * dinv_ref[...] + b_ref[...])[None]


def _k4(s0, s1, dinv, b2):
    return pl.pallas_call(
        _ep_body,
        grid=(N // BN,),
        in_specs=[
            pl.BlockSpec((BN, H), lambda i: (i, 0)),
            pl.BlockSpec((BN, H), lambda i: (i, 0)),
            pl.BlockSpec((BN, 1), lambda i: (i, 0)),
            pl.BlockSpec((1, D_MODEL), lambda i: (0, 0)),
        ],
        out_specs=pl.BlockSpec((1, BN, D_MODEL), lambda i: (0, i, 0)),
        out_shape=jax.ShapeDtypeStruct((1, N, D_MODEL), jnp.float32),
    )(s0, s1, dinv, b2)


def kernel(x, edge_index, W, b):
    src = edge_index[0]
    dst = edge_index[1]
    part = _k1(dst)
    g0, g1, dinv = _k2(x, part.T, W)
    ei2 = jnp.stack([src.reshape(E // B, B), dst.reshape(E // B, B)], axis=1)
    s0, s1 = _k3(ei2, g0, g1)
    out = _k4(s0[:N], s1[:N], dinv, b.reshape(1, D_MODEL))
    return (out, None, out)


# speedup vs baseline: 27.3221x; 1.0143x over previous
import functools

import jax
import jax.numpy as jnp
from jax import lax
from jax.experimental import pallas as pl
from jax.experimental.pallas import tpu as pltpu
from jax.experimental.pallas import tpu_sc as plsc

N = 10000
E = 320000
D_IN = 128
D_MODEL = 256
H = D_MODEL // 2

NC = 2
NS = 16
NW = NC * NS

NP = 10240
ECW = E // NW
B = 50
NBLK = E // B // NS
NQ = NBLK // 8
R = NP // NS
RCH = 128
BN = 1000

_mesh = plsc.VectorSubcoreMesh(core_axis_name="c", subcore_axis_name="s")


def _hist_body(dst_hbm, out_hbm, dst_v, hist_v):
    c = lax.axis_index("c")
    s = lax.axis_index("s")
    wid = s * NC + c

    def zero(i, _):
        hist_v[pl.ds(pl.multiple_of(i * 16, 16), 16)] = jnp.zeros((16,), jnp.float32)
        return 0

    lax.fori_loop(0, N // 16, zero, 0)

    pltpu.sync_copy(dst_hbm.at[pl.ds(wid * ECW, ECW)], dst_v)

    ones = jnp.full((16,), 1.0, jnp.float32)

    def add(i, _):
        idx = dst_v[pl.ds(pl.multiple_of(i * 16, 16), 16)]
        plsc.addupdate_scatter(hist_v, [idx], ones)
        return 0

    lax.fori_loop(0, ECW // 16, add, 0)
    pltpu.sync_copy(hist_v, out_hbm.at[wid])


_k1 = functools.partial(
    pl.kernel,
    out_type=jax.ShapeDtypeStruct((NW, N), jnp.float32),
    mesh=_mesh,
    scratch_types=[
        pltpu.VMEM((ECW,), jnp.int32),
        pltpu.VMEM((N,), jnp.float32),
    ],
    compiler_params=pltpu.CompilerParams(needs_layout_passes=False),
)(_hist_body)


def _mm_body(x_ref, part_ref, w_ref, g0_ref, g1_ref, dinv_ref):
    deg = jnp.sum(part_ref[...], axis=1) + 1.0
    dinv = lax.rsqrt(deg)
    h = jnp.dot(x_ref[...], w_ref[...], preferred_element_type=jnp.float32)
    g = h * dinv[:, None]
    g0_ref[...] = g[:, :H]
    g1_ref[...] = g[:, H:]
    dinv_ref[...] = dinv[:, None]


def _k2(x, part, W):
    return pl.pallas_call(
        _mm_body,
        grid=(N // BN,),
        in_specs=[
            pl.BlockSpec((BN, D_IN), lambda i: (i, 0)),
            pl.BlockSpec((BN, NW), lambda i: (i, 0)),
            pl.BlockSpec((D_IN, D_MODEL), lambda i: (0, 0)),
        ],
        out_specs=[
            pl.BlockSpec((BN, H), lambda i: (i, 0)),
            pl.BlockSpec((BN, H), lambda i: (i, 0)),
            pl.BlockSpec((BN, 1), lambda i: (i, 0)),
        ],
        out_shape=[
            jax.ShapeDtypeStruct((NP, H), jnp.float32),
            jax.ShapeDtypeStruct((NP, H), jnp.float32),
            jax.ShapeDtypeStruct((N, 1), jnp.float32),
        ],
    )(x, part, W)


def _scatter_body(ei2_hbm, g0_hbm, g1_hbm, out0_hbm, out1_hbm,
                  e0, e1, e2, e3, e4, e5, e6, e7,
                  r0, r1, r2, r3,
                  s_sh,
                  is0, is1, is2, is3, is4, is5, is6, is7,
                  gs0, gs1, gs2, gs3,
                  ss0, ss1, ss2, ss3):
    c = lax.axis_index("c")
    s = lax.axis_index("s")
    base = s * NBLK
    e = [e0, e1, e2, e3, e4, e5, e6, e7]
    rows = [r0, r1, r2, r3]
    isem = [is0, is1, is2, is3, is4, is5, is6, is7]
    gsem = [gs0, gs1, gs2, gs3]
    ssem = [ss0, ss1, ss2, ss3]

    def fetch_idx(k, j):
        pltpu.async_copy(ei2_hbm.at[base + k], e[j % 8], isem[j % 8])

    def wait_idx(j):
        pltpu.make_async_copy(ei2_hbm.at[0], e[j % 8], isem[j % 8]).wait()

    def start_gather(j):
        @pl.when(c == 0)
        def _():
            pltpu.async_copy(g0_hbm.at[e[j % 8].at[0]], rows[j % 4], gsem[j % 4])

        @pl.when(c == 1)
        def _():
            pltpu.async_copy(g1_hbm.at[e[j % 8].at[0]], rows[j % 4], gsem[j % 4])

    def wait_gather(j):
        pltpu.make_async_copy(g0_hbm.at[e[0].at[0]], rows[j % 4], gsem[j % 4]).wait()

    def start_scat(j):
        pltpu.async_copy(rows[j % 4], s_sh.at[e[j % 8].at[1]], ssem[j % 4], add=True)

    def wait_scat(j):
        pltpu.make_async_copy(
            rows[j % 4], s_sh.at[e[j % 8].at[1]], ssem[j % 4]).wait()

    def step(k, j, do_wait_s=True, do_next=True, do_fetch=True):
        wait_gather(j)
        start_scat(j)
        if do_wait_s:
            wait_scat(j + 2)
        if do_next:
            wait_idx(j + 2)
            start_gather(j + 2)
        if do_fetch:
            fetch_idx(k + 6, j + 6)

    for j in range(6):
        fetch_idx(j, j)

    def init(i, _):
        q = s * R + i * RCH
        sl = pl.ds(q, RCH)

        @pl.when(c == 0)
        def _():
            pltpu.sync_copy(g0_hbm.at[sl], s_sh.at[sl])

        @pl.when(c == 1)
        def _():
            pltpu.sync_copy(g1_hbm.at[sl], s_sh.at[sl])

        return 0

    lax.fori_loop(0, R // RCH, init, 0)
    plsc.subcore_barrier()

    wait_idx(0)
    start_gather(0)
    wait_idx(1)
    start_gather(1)
    for j in range(8):
        step(j, j, do_wait_s=(j >= 2))

    def body(q, _):
        k = q * 8
        for j in range(8):
            step(k + j, j)
        return 0

    lax.fori_loop(1, NQ - 1, body, 0)

    kl = NBLK - 8
    for j in range(8):
        step(kl + j, j, do_next=(kl + j + 2 < NBLK), do_fetch=(kl + j + 6 < NBLK))
    wait_scat(2)
    wait_scat(3)
    plsc.subcore_barrier()

    def wb(i, _):
        q = s * R + i * RCH
        sl = pl.ds(q, RCH)

        @pl.when(c == 0)
        def _():
            pltpu.sync_copy(s_sh.at[sl], out0_hbm.at[sl])

        @pl.when(c == 1)
        def _():
            pltpu.sync_copy(s_sh.at[sl], out1_hbm.at[sl])

        return 0

    lax.fori_loop(0, R // RCH, wb, 0)


_k3 = functools.partial(
    pl.kernel,
    out_type=[
        jax.ShapeDtypeStruct((NP, H), jnp.float32),
        jax.ShapeDtypeStruct((NP, H), jnp.float32),
    ],
    mesh=_mesh,
    scratch_types=(
        [pltpu.VMEM((2, B), jnp.int32) for _ in range(8)]
        + [pltpu.VMEM((B, H), jnp.float32) for _ in range(4)]
        + [pltpu.VMEM_SHARED((NP, H), jnp.float32)]
        + [pltpu.SemaphoreType.DMA for _ in range(16)]
    ),
    compiler_params=pltpu.CompilerParams(needs_layout_passes=False),
)(_scatter_body)


def _ep_body(s0_ref, s1_ref, dinv_ref, b_ref, out_ref):
    m = jnp.concatenate([s0_ref[...], s1_ref[...]], axis=1)
    out_ref[...] = (m * dinv_ref[...] + b_ref[...])[None]


def _k4(s0, s1, dinv, b2):
    return pl.pallas_call(
        _ep_body,
        grid=(N // BN,),
        in_specs=[
            pl.BlockSpec((BN, H), lambda i: (i, 0)),
            pl.BlockSpec((BN, H), lambda i: (i, 0)),
            pl.BlockSpec((BN, 1), lambda i: (i, 0)),
            pl.BlockSpec((1, D_MODEL), lambda i: (0, 0)),
        ],
        out_specs=pl.BlockSpec((1, BN, D_MODEL), lambda i: (0, i, 0)),
        out_shape=jax.ShapeDtypeStruct((1, N, D_MODEL), jnp.float32),
    )(s0, s1, dinv, b2)


def kernel(x, edge_index, W, b):
    src = edge_index[0]
    dst = edge_index[1]
    part = _k1(dst)
    g0, g1, dinv = _k2(x, part.T, W)
    ei2 = jnp.stack([src.reshape(E // B, B), dst.reshape(E // B, B)], axis=1)
    s0, s1 = _k3(ei2, g0, g1)
    out = _k4(s0, s1, dinv, b.reshape(1, D_MODEL))
    return (out, None, out)


# speedup vs baseline: 27.4333x; 1.0041x over previous
import functools

import jax
import jax.numpy as jnp
from jax import lax
from jax.experimental import pallas as pl
from jax.experimental.pallas import tpu as pltpu
from jax.experimental.pallas import tpu_sc as plsc

N = 10000
E = 320000
D_IN = 128
D_MODEL = 256
H = D_MODEL // 2

NC = 2
NS = 16
NW = NC * NS

NP = 10240
ECW = E // NW
B = 50
NBLK = E // B // NS
NQ = NBLK // 8
R = NP // NS
RCH = 128
BN = 1000

_mesh = plsc.VectorSubcoreMesh(core_axis_name="c", subcore_axis_name="s")


HR = NP // 128


def _hist_body(ei_flat_hbm, out_hbm, dst_v, hist_v, idx_v, deg_sh):
    c = lax.axis_index("c")
    s = lax.axis_index("s")
    wid = s * NC + c

    def zero(i, _):
        r = i >> 3
        j = i & 7
        hist_v[r, pl.ds(j * 16, 16)] = jnp.zeros((16,), jnp.float32)
        return 0

    lax.fori_loop(0, HR * 8, zero, 0)

    for j in range(HR // 16):
        idx_v[pl.ds(j * 16, 16)] = lax.iota(jnp.int32, 16) + (j * 16)

    @pl.when(s == 0)
    def _():
        pltpu.sync_copy(hist_v, deg_sh)

    pltpu.sync_copy(ei_flat_hbm.at[pl.ds(E + wid * ECW, ECW)], dst_v)
    plsc.subcore_barrier()

    ones = jnp.full((16,), 1.0, jnp.float32)

    def add(i, _):
        d = dst_v[pl.ds(pl.multiple_of(i * 16, 16), 16)]
        plsc.addupdate_scatter(hist_v, [d >> 7, d & 127], ones)
        return 0

    lax.fori_loop(0, ECW // 16, add, 0)
    pltpu.sync_copy(hist_v, deg_sh.at[idx_v], add=True)
    plsc.subcore_barrier()

    @pl.when(s < 5)
    def _():
        sl = pl.ds(s * 16, 16)
        pltpu.sync_copy(deg_sh.at[sl], out_hbm.at[c, sl])


_k1 = functools.partial(
    pl.kernel,
    out_type=jax.ShapeDtypeStruct((NC, HR, 128), jnp.float32),
    mesh=_mesh,
    scratch_types=[
        pltpu.VMEM((ECW,), jnp.int32),
        pltpu.VMEM((HR, 128), jnp.float32),
        pltpu.VMEM((HR,), jnp.int32),
        pltpu.VMEM_SHARED((HR, 128), jnp.float32),
    ],
    compiler_params=pltpu.CompilerParams(needs_layout_passes=False),
)(_hist_body)


def _mm_body(x_ref, part_ref, w_ref, g0_ref, g1_ref, dinv_ref):
    deg = jnp.sum(part_ref[...], axis=1) + 1.0
    dinv = lax.rsqrt(deg)
    h = jnp.dot(x_ref[...], w_ref[...], preferred_element_type=jnp.float32)
    g = h * dinv[:, None]
    g0_ref[...] = g[:, :H]
    g1_ref[...] = g[:, H:]
    dinv_ref[...] = dinv[:, None]


def _k2(x, part, W):
    return pl.pallas_call(
        _mm_body,
        grid=(N // BN,),
        in_specs=[
            pl.BlockSpec((BN, D_IN), lambda i: (i, 0)),
            pl.BlockSpec((BN, NC), lambda i: (i, 0)),
            pl.BlockSpec((D_IN, D_MODEL), lambda i: (0, 0)),
        ],
        out_specs=[
            pl.BlockSpec((BN, H), lambda i: (i, 0)),
            pl.BlockSpec((BN, H), lambda i: (i, 0)),
            pl.BlockSpec((BN, 1), lambda i: (i, 0)),
        ],
        out_shape=[
            jax.ShapeDtypeStruct((NP, H), jnp.float32),
            jax.ShapeDtypeStruct((NP, H), jnp.float32),
            jax.ShapeDtypeStruct((N, 1), jnp.float32),
        ],
    )(x, part, W)


def _scatter_body(ei2_hbm, g0_hbm, g1_hbm, out0_hbm, out1_hbm,
                  e0, e1, e2, e3, e4, e5, e6, e7,
                  r0, r1, r2, r3,
                  s_sh,
                  is0, is1, is2, is3, is4, is5, is6, is7,
                  gs0, gs1, gs2, gs3,
                  ss0, ss1, ss2, ss3):
    c = lax.axis_index("c")
    s = lax.axis_index("s")
    base = s * NBLK
    e = [e0, e1, e2, e3, e4, e5, e6, e7]
    rows = [r0, r1, r2, r3]
    isem = [is0, is1, is2, is3, is4, is5, is6, is7]
    gsem = [gs0, gs1, gs2, gs3]
    ssem = [ss0, ss1, ss2, ss3]

    def fetch_idx(k, j):
        pltpu.async_copy(ei2_hbm.at[base + k], e[j % 8], isem[j % 8])

    def wait_idx(j):
        pltpu.make_async_copy(ei2_hbm.at[0], e[j % 8], isem[j % 8]).wait()

    def start_gather(j):
        @pl.when(c == 0)
        def _():
            pltpu.async_copy(g0_hbm.at[e[j % 8].at[0]], rows[j % 4], gsem[j % 4])

        @pl.when(c == 1)
        def _():
            pltpu.async_copy(g1_hbm.at[e[j % 8].at[0]], rows[j % 4], gsem[j % 4])

    def wait_gather(j):
        pltpu.make_async_copy(g0_hbm.at[e[0].at[0]], rows[j % 4], gsem[j % 4]).wait()

    def start_scat(j):
        pltpu.async_copy(rows[j % 4], s_sh.at[e[j % 8].at[1]], ssem[j % 4], add=True)

    def wait_scat(j):
        pltpu.make_async_copy(
            rows[j % 4], s_sh.at[e[j % 8].at[1]], ssem[j % 4]).wait()

    def step(k, j, do_wait_s=True, do_next=True, do_fetch=True):
        wait_gather(j)
        start_scat(j)
        if do_wait_s:
            wait_scat(j + 2)
        if do_next:
            wait_idx(j + 2)
            start_gather(j + 2)
        if do_fetch:
            fetch_idx(k + 6, j + 6)

    for j in range(6):
        fetch_idx(j, j)

    def init(i, _):
        q = s * R + i * RCH
        sl = pl.ds(q, RCH)

        @pl.when(c == 0)
        def _():
            pltpu.sync_copy(g0_hbm.at[sl], s_sh.at[sl])

        @pl.when(c == 1)
        def _():
            pltpu.sync_copy(g1_hbm.at[sl], s_sh.at[sl])

        return 0

    lax.fori_loop(0, R // RCH, init, 0)
    plsc.subcore_barrier()

    wait_idx(0)
    start_gather(0)
    wait_idx(1)
    start_gather(1)
    for j in range(8):
        step(j, j, do_wait_s=(j >= 2))

    def body(q, _):
        k = q * 8
        for j in range(8):
            step(k + j, j)
        return 0

    lax.fori_loop(1, NQ - 1, body, 0)

    kl = NBLK - 8
    for j in range(8):
        step(kl + j, j, do_next=(kl + j + 2 < NBLK), do_fetch=(kl + j + 6 < NBLK))
    wait_scat(2)
    wait_scat(3)
    plsc.subcore_barrier()

    def wb(i, _):
        q = s * R + i * RCH
        sl = pl.ds(q, RCH)

        @pl.when(c == 0)
        def _():
            pltpu.sync_copy(s_sh.at[sl], out0_hbm.at[sl])

        @pl.when(c == 1)
        def _():
            pltpu.sync_copy(s_sh.at[sl], out1_hbm.at[sl])

        return 0

    lax.fori_loop(0, R // RCH, wb, 0)


_k3 = functools.partial(
    pl.kernel,
    out_type=[
        jax.ShapeDtypeStruct((NP, H), jnp.float32),
        jax.ShapeDtypeStruct((NP, H), jnp.float32),
    ],
    mesh=_mesh,
    scratch_types=(
        [pltpu.VMEM((2, B), jnp.int32) for _ in range(8)]
        + [pltpu.VMEM((B, H), jnp.float32) for _ in range(4)]
        + [pltpu.VMEM_SHARED((NP, H), jnp.float32)]
        + [pltpu.SemaphoreType.DMA for _ in range(16)]
    ),
    compiler_params=pltpu.CompilerParams(needs_layout_passes=False),
)(_scatter_body)


def _ep_body(s0_ref, s1_ref, dinv_ref, b_ref, out_ref, out2_ref):
    m = jnp.concatenate([s0_ref[...], s1_ref[...]], axis=1)
    o = (m * dinv_ref[...] + b_ref[...])[None]
    out_ref[...] = o
    out2_ref[...] = o


def _k4(s0, s1, dinv, b2):
    return pl.pallas_call(
        _ep_body,
        grid=(N // BN,),
        in_specs=[
            pl.BlockSpec((BN, H), lambda i: (i, 0)),
            pl.BlockSpec((BN, H), lambda i: (i, 0)),
            pl.BlockSpec((BN, 1), lambda i: (i, 0)),
            pl.BlockSpec((1, D_MODEL), lambda i: (0, 0)),
        ],
        out_specs=[
            pl.BlockSpec((1, BN, D_MODEL), lambda i: (0, i, 0)),
            pl.BlockSpec((1, BN, D_MODEL), lambda i: (0, i, 0)),
        ],
        out_shape=[
            jax.ShapeDtypeStruct((1, N, D_MODEL), jnp.float32),
            jax.ShapeDtypeStruct((1, N, D_MODEL), jnp.float32),
        ],
    )(s0, s1, dinv, b2)


def kernel(x, edge_index, W, b):
    part = _k1(edge_index.reshape(2 * E))
    part_t = part.reshape(NC, NP)[:, :N].T
    g0, g1, dinv = _k2(x, part_t, W)
    ei2 = edge_index.reshape(2, E // B, B).transpose(1, 0, 2)
    s0, s1 = _k3(ei2, g0, g1)
    out, out2 = _k4(s0, s1, dinv, b.reshape(1, D_MODEL))
    return (out, None, out2)


# speedup vs baseline: 32.6233x; 1.1892x over previous
import functools

import jax
import jax.numpy as jnp
from jax import lax
from jax.experimental import pallas as pl
from jax.experimental.pallas import tpu as pltpu
from jax.experimental.pallas import tpu_sc as plsc

N = 10000
E = 320000
D_IN = 128
D_MODEL = 256
H = D_MODEL // 2

NC = 2
NS = 16
NW = NC * NS

NP = 10240
ECW = E // NW
B = 80
NBLK = E // B // NS
NQ = NBLK // 8
R = NP // NS
RCH = 128
BN = 1000

_mesh = plsc.VectorSubcoreMesh(core_axis_name="c", subcore_axis_name="s")


HR = NP // 128


def _hist_body(ei_flat_hbm, out_hbm, dst_v, hist_v, idx_v, deg_sh):
    c = lax.axis_index("c")
    s = lax.axis_index("s")
    wid = s * NC + c

    def zero(i, _):
        r = i >> 3
        j = i & 7
        hist_v[r, pl.ds(j * 16, 16)] = jnp.zeros((16,), jnp.float32)
        return 0

    lax.fori_loop(0, HR * 8, zero, 0)

    for j in range(HR // 16):
        idx_v[pl.ds(j * 16, 16)] = lax.iota(jnp.int32, 16) + (j * 16)

    @pl.when(s == 0)
    def _():
        pltpu.sync_copy(hist_v, deg_sh)

    pltpu.sync_copy(ei_flat_hbm.at[pl.ds(E + wid * ECW, ECW)], dst_v)
    plsc.subcore_barrier()

    ones = jnp.full((16,), 1.0, jnp.float32)

    def add(i, _):
        d = dst_v[pl.ds(pl.multiple_of(i * 16, 16), 16)]
        plsc.addupdate_scatter(hist_v, [d >> 7, d & 127], ones)
        return 0

    lax.fori_loop(0, ECW // 16, add, 0)
    pltpu.sync_copy(hist_v, deg_sh.at[idx_v], add=True)
    plsc.subcore_barrier()

    @pl.when(s < 5)
    def _():
        sl = pl.ds(s * 16, 16)
        pltpu.sync_copy(deg_sh.at[sl], out_hbm.at[c, sl])


_k1 = functools.partial(
    pl.kernel,
    out_type=jax.ShapeDtypeStruct((NC, HR, 128), jnp.float32),
    mesh=_mesh,
    scratch_types=[
        pltpu.VMEM((ECW,), jnp.int32),
        pltpu.VMEM((HR, 128), jnp.float32),
        pltpu.VMEM((HR,), jnp.int32),
        pltpu.VMEM_SHARED((HR, 128), jnp.float32),
    ],
    compiler_params=pltpu.CompilerParams(needs_layout_passes=False),
)(_hist_body)


def _mma_body(x_ref, w_ref, h_ref):
    h_ref[...] = jnp.dot(x_ref[...], w_ref[...],
                         preferred_element_type=jnp.float32)


def _k2a(x, W):
    return pl.pallas_call(
        _mma_body,
        grid=(N // BN,),
        in_specs=[
            pl.BlockSpec((BN, D_IN), lambda i: (i, 0)),
            pl.BlockSpec((D_IN, D_MODEL), lambda i: (0, 0)),
        ],
        out_specs=pl.BlockSpec((BN, D_MODEL), lambda i: (i, 0)),
        out_shape=jax.ShapeDtypeStruct((N, D_MODEL), jnp.float32),
    )(x, W)


def _mmb_body(h_ref, part_ref, g0_ref, g1_ref, dinv_ref):
    deg = jnp.sum(part_ref[...], axis=1) + 1.0
    dinv = lax.rsqrt(deg)
    g = h_ref[...] * dinv[:, None]
    g0_ref[...] = g[:, :H]
    g1_ref[...] = g[:, H:]
    dinv_ref[...] = dinv[:, None]


def _k2b(h, part):
    return pl.pallas_call(
        _mmb_body,
        grid=(N // BN,),
        in_specs=[
            pl.BlockSpec((BN, D_MODEL), lambda i: (i, 0)),
            pl.BlockSpec((BN, NC), lambda i: (i, 0)),
        ],
        out_specs=[
            pl.BlockSpec((BN, H), lambda i: (i, 0)),
            pl.BlockSpec((BN, H), lambda i: (i, 0)),
            pl.BlockSpec((BN, 1), lambda i: (i, 0)),
        ],
        out_shape=[
            jax.ShapeDtypeStruct((NP, H), jnp.float32),
            jax.ShapeDtypeStruct((NP, H), jnp.float32),
            jax.ShapeDtypeStruct((N, 1), jnp.float32),
        ],
    )(h, part)


def _scatter_body(ei_flat_hbm, g0_hbm, g1_hbm, out0_hbm, out1_hbm,
                  e0, e1, e2, e3, e4, e5, e6, e7,
                  r0, r1, r2, r3,
                  s_sh,
                  is0, is1, is2, is3, is4, is5, is6, is7,
                  gs0, gs1, gs2, gs3,
                  ss0, ss1, ss2, ss3):
    c = lax.axis_index("c")
    s = lax.axis_index("s")
    base = s * NBLK
    e = [e0, e1, e2, e3, e4, e5, e6, e7]
    rows = [r0, r1, r2, r3]
    isem = [is0, is1, is2, is3, is4, is5, is6, is7]
    gsem = [gs0, gs1, gs2, gs3]
    ssem = [ss0, ss1, ss2, ss3]

    def fetch_idx(k, j):
        off = (base + k) * B
        pltpu.async_copy(ei_flat_hbm.at[pl.ds(off, B)], e[j % 8].at[0],
                         isem[j % 8])
        pltpu.async_copy(ei_flat_hbm.at[pl.ds(E + off, B)], e[j % 8].at[1],
                         isem[j % 8])

    def wait_idx(j):
        pltpu.make_async_copy(
            ei_flat_hbm.at[pl.ds(0, B)], e[j % 8].at[0], isem[j % 8]).wait()
        pltpu.make_async_copy(
            ei_flat_hbm.at[pl.ds(0, B)], e[j % 8].at[1], isem[j % 8]).wait()

    def start_gather(j):
        @pl.when(c == 0)
        def _():
            pltpu.async_copy(g0_hbm.at[e[j % 8].at[0]], rows[j % 4], gsem[j % 4])

        @pl.when(c == 1)
        def _():
            pltpu.async_copy(g1_hbm.at[e[j % 8].at[0]], rows[j % 4], gsem[j % 4])

    def wait_gather(j):
        pltpu.make_async_copy(g0_hbm.at[e[0].at[0]], rows[j % 4], gsem[j % 4]).wait()

    def start_scat(j):
        pltpu.async_copy(rows[j % 4], s_sh.at[e[j % 8].at[1]], ssem[j % 4], add=True)

    def wait_scat(j):
        pltpu.make_async_copy(
            rows[j % 4], s_sh.at[e[j % 8].at[1]], ssem[j % 4]).wait()

    def step(k, j, do_wait_s=True, do_next=True, do_fetch=True):
        wait_gather(j)
        start_scat(j)
        if do_wait_s:
            wait_scat(j + 2)
        if do_next:
            wait_idx(j + 2)
            start_gather(j + 2)
        if do_fetch:
            fetch_idx(k + 6, j + 6)

    for j in range(6):
        fetch_idx(j, j)

    def init(i, _):
        q = s * R + i * RCH
        sl = pl.ds(q, RCH)

        @pl.when(c == 0)
        def _():
            pltpu.sync_copy(g0_hbm.at[sl], s_sh.at[sl])

        @pl.when(c == 1)
        def _():
            pltpu.sync_copy(g1_hbm.at[sl], s_sh.at[sl])

        return 0

    lax.fori_loop(0, R // RCH, init, 0)
    plsc.subcore_barrier()

    wait_idx(0)
    start_gather(0)
    wait_idx(1)
    start_gather(1)
    for j in range(8):
        step(j, j, do_wait_s=(j >= 2))

    def body(q, _):
        k = q * 8
        for j in range(8):
            step(k + j, j)
        return 0

    lax.fori_loop(1, (NBLK - 10) // 8, body, 0)

    kl = NBLK - 10
    for j in range(10):
        step(kl + j, j, do_next=(kl + j + 2 < NBLK), do_fetch=(kl + j + 6 < NBLK))
    wait_scat(NBLK - 2)
    wait_scat(NBLK - 1)
    plsc.subcore_barrier()

    def wb(i, _):
        q = s * R + i * RCH
        sl = pl.ds(q, RCH)

        @pl.when(c == 0)
        def _():
            pltpu.sync_copy(s_sh.at[sl], out0_hbm.at[sl])

        @pl.when(c == 1)
        def _():
            pltpu.sync_copy(s_sh.at[sl], out1_hbm.at[sl])

        return 0

    lax.fori_loop(0, R // RCH, wb, 0)


_k3 = functools.partial(
    pl.kernel,
    out_type=[
        jax.ShapeDtypeStruct((NP, H), jnp.float32),
        jax.ShapeDtypeStruct((NP, H), jnp.float32),
    ],
    mesh=_mesh,
    scratch_types=(
        [pltpu.VMEM((2, B), jnp.int32) for _ in range(8)]
        + [pltpu.VMEM((B, H), jnp.float32) for _ in range(4)]
        + [pltpu.VMEM_SHARED((NP, H), jnp.float32)]
        + [pltpu.SemaphoreType.DMA for _ in range(16)]
    ),
    compiler_params=pltpu.CompilerParams(needs_layout_passes=False),
)(_scatter_body)


def _ep_body(s0_ref, s1_ref, dinv_ref, b_ref, out_ref, out2_ref):
    m = jnp.concatenate([s0_ref[...], s1_ref[...]], axis=1)
    o = (m * dinv_ref[...] + b_ref[...])[None]
    out_ref[...] = o
    out2_ref[...] = o


def _k4(s0, s1, dinv, b2):
    return pl.pallas_call(
        _ep_body,
        grid=(N // BN,),
        in_specs=[
            pl.BlockSpec((BN, H), lambda i: (i, 0)),
            pl.BlockSpec((BN, H), lambda i: (i, 0)),
            pl.BlockSpec((BN, 1), lambda i: (i, 0)),
            pl.BlockSpec((1, D_MODEL), lambda i: (0, 0)),
        ],
        out_specs=[
            pl.BlockSpec((1, BN, D_MODEL), lambda i: (0, i, 0)),
            pl.BlockSpec((1, BN, D_MODEL), lambda i: (0, i, 0)),
        ],
        out_shape=[
            jax.ShapeDtypeStruct((1, N, D_MODEL), jnp.float32),
            jax.ShapeDtypeStruct((1, N, D_MODEL), jnp.float32),
        ],
    )(s0, s1, dinv, b2)


def kernel(x, edge_index, W, b):
    ei_flat = edge_index.reshape(2 * E)
    h = _k2a(x, W)
    part = _k1(ei_flat)
    part_t = part.reshape(NC, NP)[:, :N].T
    g0, g1, dinv = _k2b(h, part_t)
    s0, s1 = _k3(ei_flat, g0, g1)
    out, out2 = _k4(s0, s1, dinv, b.reshape(1, D_MODEL))
    return (out, None, out2)
